# baseline (speedup 1.0000x reference)
import functools

import jax
import jax.numpy as jnp
from jax import lax
from jax.experimental import pallas as pl
from jax.experimental.pallas import tpu as pltpu
from jax.experimental.pallas import tpu_sc as plsc

B, A, H, D, K = 4096, 64, 1024, 256, 8192
BETA = 0.25
BB = 256
GRID = B // BB


def _dot(x, y):
    return jnp.dot(x, y, preferred_element_type=jnp.float32)


def _row_sq_norm(e2):
    parts = []
    for s in range(8):
        a = e2[:, s:s + 1]
        for t in range(1, 32):
            a = a + e2[:, t * 8 + s:t * 8 + s + 1]
        parts.append(a)
    b4 = [parts[s] + parts[s + 4] for s in range(4)]
    b2 = [b4[s] + b4[s + 2] for s in range(2)]
    return b2[0] + b2[1]


def _enc_body(a_ref, w1_ref, b1_ref, w2_ref, b2_ref, w3_ref, b3_ref, cb_ref,
              enc_ref, idx_ref):
    a = a_ref[...]
    h = jnp.maximum(_dot(a, w1_ref[...]) + b1_ref[...], 0.0)
    h = jnp.maximum(_dot(h.astype(jnp.bfloat16), w2_ref[...]) + b2_ref[...],
                    0.0)
    enc = _dot(h, w3_ref[...]) + b3_ref[...]
    enc_ref[...] = enc
    cb = cb_ref[...]
    dot = _dot(enc.astype(jnp.bfloat16), cb.astype(jnp.bfloat16).T)
    s_e = _row_sq_norm(enc * enc)
    s_c = jnp.sum(cb * cb, axis=1)[None, :]
    d2 = (s_e + s_c) - 2.0 * dot
    dist = jnp.sqrt(jnp.maximum(d2, 0.0))
    minv = jnp.min(dist, axis=1, keepdims=True)
    jidx = lax.broadcasted_iota(jnp.int32, dist.shape, 1)
    idx_ref[...] = jnp.min(jnp.where(dist == minv, jidx, K), axis=1,
                           keepdims=True)


def _encode_argmin(action, w1, b1, w2, b2, w3, b3, cb):
    full = lambda i: (0, 0)
    return pl.pallas_call(
        _enc_body,
        grid=(GRID,),
        in_specs=[
            pl.BlockSpec((BB, A), lambda i: (i, 0)),
            pl.BlockSpec((A, H), full),
            pl.BlockSpec((1, H), full),
            pl.BlockSpec((H, H), full),
            pl.BlockSpec((1, H), full),
            pl.BlockSpec((H, D), full),
            pl.BlockSpec((1, D), full),
            pl.BlockSpec((K, D), full),
        ],
        out_specs=[
            pl.BlockSpec((BB, D), lambda i: (i, 0)),
            pl.BlockSpec((BB, 1), lambda i: (i, 0)),
        ],
        out_shape=[
            jax.ShapeDtypeStruct((B, D), jnp.float32),
            jax.ShapeDtypeStruct((B, 1), jnp.int32),
        ],
    )(action, w1, b1, w2, b2, w3, b3, cb)


_NC, _NS = 2, 16
_NW = _NC * _NS
_BPW = B // _NW


def _gather_body(table_hbm, idx_hbm, out_hbm, idx_v, rows_v, sem):
    wid = lax.axis_index("s") * _NC + lax.axis_index("c")
    base = wid * _BPW
    pltpu.sync_copy(idx_hbm.at[pl.ds(base, _BPW)], idx_v)
    pltpu.async_copy(table_hbm.at[idx_v], rows_v, sem).wait()
    pltpu.sync_copy(rows_v, out_hbm.at[pl.ds(base, _BPW)])


def _sc_gather(codebook, idx):
    gather = functools.partial(
        pl.kernel,
        out_type=jax.ShapeDtypeStruct((B, D), jnp.float32),
        mesh=plsc.VectorSubcoreMesh(core_axis_name="c", subcore_axis_name="s"),
        scratch_types=[
            pltpu.VMEM((_BPW,), jnp.int32),
            pltpu.VMEM((_BPW, D), jnp.float32),
            pltpu.SemaphoreType.DMA,
        ],
    )(_gather_body)
    return gather(codebook, idx)


def _dec_body(q_ref, enc_ref, w1_ref, b1_ref, w2_ref, b2_ref, hw_ref, hb_ref,
              out_ref, loss_ref, acc_ref):
    i = pl.program_id(0)
    q = q_ref[...]
    dec = jnp.maximum(_dot(q, w1_ref[...]) + b1_ref[...], 0.0)
    dec = jnp.maximum(_dot(dec.astype(jnp.bfloat16), w2_ref[...])
                      + b2_ref[...], 0.0)
    out_ref[...] = jnp.tanh(_dot(dec, hw_ref[...]) + hb_ref[...])
    diff = enc_ref[...] - q
    part = jnp.sum(diff * diff)

    @pl.when(i == 0)
    def _():
        acc_ref[0] = part

    @pl.when(i > 0)
    def _():
        acc_ref[0] = acc_ref[0] + part

    @pl.when(i == pl.num_programs(0) - 1)
    def _():
        loss_ref[0, 0] = acc_ref[0] * ((1.0 + BETA) / (B * D))


def _decode(q, enc, w1, b1, w2, b2, hw, hb):
    full = lambda i: (0, 0)
    return pl.pallas_call(
        _dec_body,
        grid=(GRID,),
        in_specs=[
            pl.BlockSpec((BB, D), lambda i: (i, 0)),
            pl.BlockSpec((BB, D), lambda i: (i, 0)),
            pl.BlockSpec((D, H), full),
            pl.BlockSpec((1, H), full),
            pl.BlockSpec((H, H), full),
            pl.BlockSpec((1, H), full),
            pl.BlockSpec((H, A), full),
            pl.BlockSpec((1, A), full),
        ],
        out_specs=[
            pl.BlockSpec((BB, A), lambda i: (i, 0)),
            pl.BlockSpec(memory_space=pltpu.SMEM),
        ],
        out_shape=[
            jax.ShapeDtypeStruct((B, A), jnp.float32),
            jax.ShapeDtypeStruct((1, 1), jnp.float32),
        ],
        scratch_shapes=[pltpu.SMEM((1,), jnp.float32)],
    )(q, enc, w1, b1, w2, b2, hw, hb)


def kernel(action, enc_w1, enc_b1, enc_w2, enc_b2, enc_w3, enc_b3, codebook,
           dec_w1, dec_b1, dec_w2, dec_b2, head_w, head_b):
    enc, idx2d = _encode_argmin(
        action, enc_w1, enc_b1.reshape(1, H), enc_w2, enc_b2.reshape(1, H),
        enc_w3, enc_b3.reshape(1, D), codebook)
    idx = idx2d.reshape(B)
    quantized = _sc_gather(codebook, idx)
    recons, loss2d = _decode(
        quantized, enc, dec_w1, dec_b1.reshape(1, H), dec_w2,
        dec_b2.reshape(1, H), head_w, head_b.reshape(1, A))
    return recons, idx, loss2d.reshape(())
